# baseline (speedup 1.0000x reference)
import functools

import jax
import jax.numpy as jnp
from jax import lax
from jax.experimental import pallas as pl
from jax.experimental.pallas import tpu as pltpu
from jax.experimental.pallas import tpu_sc as plsc

_N_CORES = 2
_N_SUBCORES = 16
_NW = _N_CORES * _N_SUBCORES
_LANES = 16

_IDS_PER_ROW = 128
_CR = 4
_CHUNK = _CR * _IDS_PER_ROW
_NBUF = 2


def _proj_body(emb_ref, w1a_ref, w1b_ref, b1_ref, ps_ref, pd_ref):
    e = emb_ref[...]
    ps_ref[...] = (
        jnp.dot(e, w1a_ref[...], preferred_element_type=jnp.float32) + b1_ref[...]
    ).astype(jnp.bfloat16)
    pd_ref[...] = jnp.dot(
        e, w1b_ref[...], preferred_element_type=jnp.float32).astype(jnp.bfloat16)


def _node_projections(emb, W1, b1):
    n, d = emb.shape
    h = W1.shape[1]
    return pl.pallas_call(
        _proj_body,
        out_shape=(
            jax.ShapeDtypeStruct((n, h), jnp.bfloat16),
            jax.ShapeDtypeStruct((n, h), jnp.bfloat16),
        ),
    )(emb, W1[:d], W1[d:], b1.reshape(1, h))


def _edge_body(ps_hbm, pd_hbm, w2b_hbm, src_hbm, dst_hbm, out_hbm,
               sidx_all, didx_all, srows, drows, outv, w2v, tbuf, gsems, osems):
    wid = lax.axis_index("s") * _N_CORES + lax.axis_index("c")
    rpw = src_hbm.shape[0] // _NW
    nchunks = rpw // _CR
    h = ps_hbm.shape[1]
    nk = h // (2 * _LANES)
    pltpu.sync_copy(src_hbm.at[pl.ds(wid * rpw, rpw)], sidx_all)
    pltpu.sync_copy(dst_hbm.at[pl.ds(wid * rpw, rpw)], didx_all)
    pltpu.sync_copy(w2b_hbm, w2v)
    w2k = [w2v[pl.ds(_LANES * k, _LANES)] for k in range(h // _LANES)]
    w2pk = [plsc.pack(w2k[2 * k], w2k[2 * k + 1],
                      format=plsc.PackFormat.INTERLEAVED) for k in range(nk)]
    b2vec = w2v[pl.ds(h, _LANES)]
    col_idx = [lax.iota(jnp.int32, _LANES) * _LANES + l for l in range(_LANES)]

    def fetch(c, b):
        for j in range(_CR):
            dst_sl = pl.ds(j * _IDS_PER_ROW, _IDS_PER_ROW)
            pltpu.async_copy(ps_hbm.at[sidx_all.at[c * _CR + j]],
                             srows.at[b, dst_sl], gsems.at[b])
            pltpu.async_copy(pd_hbm.at[didx_all.at[c * _CR + j]],
                             drows.at[b, dst_sl], gsems.at[b])

    def wait_gathers(b):
        for _ in range(2 * _CR):
            pltpu.make_async_copy(
                ps_hbm.at[pl.ds(0, _IDS_PER_ROW)],
                srows.at[b, pl.ds(0, _IDS_PER_ROW)], gsems.at[b]).wait()

    def compute(c, b, out_pending):
        def group_body(g, gcarry):
            for q in range(_LANES // 4):
                accs = [None] * 4
                for k in range(nk):
                    for i in range(4):
                        e = g * _LANES + q * 4 + i
                        s = srows[b, e, pl.ds(2 * _LANES * k, 2 * _LANES)]
                        dd = drows[b, e, pl.ds(2 * _LANES * k, 2 * _LANES)]
                        t = jnp.maximum(s + dd, 0.0) * w2pk[k]
                        accs[i] = t if k == 0 else accs[i] + t
                for i in range(4):
                    pa, pb = plsc.unpack(
                        accs[i], format=plsc.PackFormat.INTERLEAVED,
                        preferred_element_type=jnp.float32)
                    tbuf[pl.ds((q * 4 + i) * _LANES, _LANES)] = pa + pb
            res = b2vec
            for l in range(_LANES):
                res = res + plsc.load_gather(tbuf, [col_idx[l]])
            outv[b, pl.ds(g * _LANES, _LANES)] = res
            return gcarry

        @pl.when(out_pending)
        def _():
            pltpu.make_async_copy(
                outv.at[b], out_hbm.at[pl.ds(0, _CHUNK)], osems.at[b]).wait()

        lax.fori_loop(0, _CHUNK // _LANES, group_body, 0)
        row0 = wid * rpw + c * _CR
        pltpu.async_copy(outv.at[b],
                         out_hbm.at[pl.ds(row0 * _IDS_PER_ROW, _CHUNK)],
                         osems.at[b])

    cmax = nchunks - 1
    fetch(0, 0)

    def pair_body(p, carry):
        c0 = 2 * p
        fetch(jnp.minimum(c0 + 1, cmax), 1)
        wait_gathers(0)
        compute(c0, 0, p > 0)
        fetch(jnp.minimum(c0 + 2, cmax), 0)
        wait_gathers(1)
        compute(c0 + 1, 1, p > 0)
        return carry

    lax.fori_loop(0, nchunks // 2, pair_body, 0)
    for b in range(_NBUF):
        pltpu.make_async_copy(
            outv.at[b], out_hbm.at[pl.ds(0, _CHUNK)], osems.at[b]).wait()


def kernel(emb, edges_src_ids, edges_dst_ids, W1, b1, W2, b2):
    e_total = edges_src_ids.shape[0]
    h = W1.shape[1]

    ps, pd = _node_projections(emb, W1, b1)

    grain = _NW * _CHUNK
    e_pad = ((e_total + grain - 1) // grain) * grain
    pad = e_pad - e_total
    src2d = jnp.pad(edges_src_ids.astype(jnp.int32), (0, pad)).reshape(-1, _IDS_PER_ROW)
    dst2d = jnp.pad(edges_dst_ids.astype(jnp.int32), (0, pad)).reshape(-1, _IDS_PER_ROW)

    w2col = W2[:, 0]
    w2parts = []
    for k in range(h // (2 * _LANES)):
        blk = w2col[2 * _LANES * k:2 * _LANES * (k + 1)]
        w2parts += [blk[0::2], blk[1::2]]
    w2b = jnp.concatenate(w2parts + [jnp.broadcast_to(b2, (_LANES,))])

    rpw = e_pad // (_IDS_PER_ROW * _NW)
    edge_fn = functools.partial(
        pl.kernel,
        out_type=jax.ShapeDtypeStruct((e_pad,), jnp.float32),
        mesh=plsc.VectorSubcoreMesh(
            core_axis_name="c", subcore_axis_name="s",
            num_cores=_N_CORES, num_subcores=_N_SUBCORES),
        scratch_types=[
            pltpu.VMEM((rpw, _IDS_PER_ROW), jnp.int32),
            pltpu.VMEM((rpw, _IDS_PER_ROW), jnp.int32),
            pltpu.VMEM((_NBUF, _CHUNK, h), jnp.bfloat16),
            pltpu.VMEM((_NBUF, _CHUNK, h), jnp.bfloat16),
            pltpu.VMEM((_NBUF, _CHUNK), jnp.float32),
            pltpu.VMEM((h + _LANES,), jnp.float32),
            pltpu.VMEM((_LANES * _LANES,), jnp.float32),
            pltpu.SemaphoreType.DMA((_NBUF,)),
            pltpu.SemaphoreType.DMA((_NBUF,)),
        ],
        compiler_params=pltpu.CompilerParams(
            needs_layout_passes=False, use_tc_tiling_on_sc=False),
    )(_edge_body)

    out = edge_fn(ps, pd, w2b, src2d, dst2d)
    return out[:e_total].reshape(e_total, 1)


# speedup vs baseline: 1.4114x; 1.4114x over previous
import functools

import jax
import jax.numpy as jnp
from jax import lax
from jax.experimental import pallas as pl
from jax.experimental.pallas import tpu as pltpu
from jax.experimental.pallas import tpu_sc as plsc

_N_CORES = 2
_N_SUBCORES = 16
_NW = _N_CORES * _N_SUBCORES
_LANES = 16

_IDS_PER_ROW = 128
_CR = 4
_CHUNK = _CR * _IDS_PER_ROW
_NBUF = 2


def _proj_body(emb_ref, w1a_ref, w1b_ref, b1_ref, ps_ref, pd_ref):
    e = emb_ref[...]
    ps_ref[...] = (
        jnp.dot(e, w1a_ref[...], preferred_element_type=jnp.float32) + b1_ref[...]
    ).astype(jnp.bfloat16)
    pd_ref[...] = jnp.dot(
        e, w1b_ref[...], preferred_element_type=jnp.float32).astype(jnp.bfloat16)


def _node_projections(emb, W1, b1):
    n, d = emb.shape
    h = W1.shape[1]
    return pl.pallas_call(
        _proj_body,
        out_shape=(
            jax.ShapeDtypeStruct((n, h), jnp.bfloat16),
            jax.ShapeDtypeStruct((n, h), jnp.bfloat16),
        ),
    )(emb, W1[:d], W1[d:], b1.reshape(1, h))


def _edge_body(ps_hbm, pd_hbm, w2b_hbm, src_hbm, dst_hbm, out_hbm,
               sidx_all, didx_all, srows, drows, outv, w2v, tbuf, ps_sh, pd_sh,
               gsems, osems):
    sid = lax.axis_index("s")
    wid = sid * _N_CORES + lax.axis_index("c")
    rpw = src_hbm.shape[0] // _NW
    nchunks = rpw // _CR
    h = ps_hbm.shape[1]
    nk = h // (2 * _LANES)
    n_nodes = ps_hbm.shape[0]
    stripe = n_nodes // _N_SUBCORES
    pltpu.sync_copy(ps_hbm.at[pl.ds(sid * stripe, stripe)],
                    ps_sh.at[pl.ds(sid * stripe, stripe)])
    pltpu.sync_copy(pd_hbm.at[pl.ds(sid * stripe, stripe)],
                    pd_sh.at[pl.ds(sid * stripe, stripe)])
    pltpu.sync_copy(src_hbm.at[pl.ds(wid * rpw, rpw)], sidx_all)
    pltpu.sync_copy(dst_hbm.at[pl.ds(wid * rpw, rpw)], didx_all)
    pltpu.sync_copy(w2b_hbm, w2v)
    plsc.subcore_barrier()
    w2k = [w2v[pl.ds(_LANES * k, _LANES)] for k in range(h // _LANES)]
    w2pk = [plsc.pack(w2k[2 * k], w2k[2 * k + 1],
                      format=plsc.PackFormat.INTERLEAVED) for k in range(nk)]
    b2vec = w2v[pl.ds(h, _LANES)]
    col_idx = [lax.iota(jnp.int32, _LANES) * _LANES + l for l in range(_LANES)]

    def fetch(c, b):
        for j in range(_CR):
            dst_sl = pl.ds(j * _IDS_PER_ROW, _IDS_PER_ROW)
            pltpu.async_copy(ps_sh.at[sidx_all.at[c * _CR + j]],
                             srows.at[b, dst_sl], gsems.at[b])
            pltpu.async_copy(pd_sh.at[didx_all.at[c * _CR + j]],
                             drows.at[b, dst_sl], gsems.at[b])

    def wait_gathers(b):
        for _ in range(2 * _CR):
            pltpu.make_async_copy(
                ps_hbm.at[pl.ds(0, _IDS_PER_ROW)],
                srows.at[b, pl.ds(0, _IDS_PER_ROW)], gsems.at[b]).wait()

    def compute(c, b, out_pending):
        def group_body(g, gcarry):
            for q in range(_LANES // 4):
                accs = [None] * 4
                for k in range(nk):
                    for i in range(4):
                        e = g * _LANES + q * 4 + i
                        s = srows[b, e, pl.ds(2 * _LANES * k, 2 * _LANES)]
                        dd = drows[b, e, pl.ds(2 * _LANES * k, 2 * _LANES)]
                        t = jnp.maximum(s + dd, 0.0) * w2pk[k]
                        accs[i] = t if k == 0 else accs[i] + t
                for i in range(4):
                    pa, pb = plsc.unpack(
                        accs[i], format=plsc.PackFormat.INTERLEAVED,
                        preferred_element_type=jnp.float32)
                    tbuf[pl.ds((q * 4 + i) * _LANES, _LANES)] = pa + pb
            res = b2vec
            for l in range(_LANES):
                res = res + plsc.load_gather(tbuf, [col_idx[l]])
            outv[b, pl.ds(g * _LANES, _LANES)] = res
            return gcarry

        @pl.when(out_pending)
        def _():
            pltpu.make_async_copy(
                outv.at[b], out_hbm.at[pl.ds(0, _CHUNK)], osems.at[b]).wait()

        lax.fori_loop(0, _CHUNK // _LANES, group_body, 0)
        row0 = wid * rpw + c * _CR
        pltpu.async_copy(outv.at[b],
                         out_hbm.at[pl.ds(row0 * _IDS_PER_ROW, _CHUNK)],
                         osems.at[b])

    cmax = nchunks - 1
    fetch(0, 0)

    def pair_body(p, carry):
        c0 = 2 * p
        fetch(jnp.minimum(c0 + 1, cmax), 1)
        wait_gathers(0)
        compute(c0, 0, p > 0)
        fetch(jnp.minimum(c0 + 2, cmax), 0)
        wait_gathers(1)
        compute(c0 + 1, 1, p > 0)
        return carry

    lax.fori_loop(0, nchunks // 2, pair_body, 0)
    for b in range(_NBUF):
        pltpu.make_async_copy(
            outv.at[b], out_hbm.at[pl.ds(0, _CHUNK)], osems.at[b]).wait()


def kernel(emb, edges_src_ids, edges_dst_ids, W1, b1, W2, b2):
    e_total = edges_src_ids.shape[0]
    h = W1.shape[1]

    ps, pd = _node_projections(emb, W1, b1)

    grain = _NW * _CHUNK
    e_pad = ((e_total + grain - 1) // grain) * grain
    pad = e_pad - e_total
    src2d = jnp.pad(edges_src_ids.astype(jnp.int32), (0, pad)).reshape(-1, _IDS_PER_ROW)
    dst2d = jnp.pad(edges_dst_ids.astype(jnp.int32), (0, pad)).reshape(-1, _IDS_PER_ROW)

    w2col = W2[:, 0]
    w2parts = []
    for k in range(h // (2 * _LANES)):
        blk = w2col[2 * _LANES * k:2 * _LANES * (k + 1)]
        w2parts += [blk[0::2], blk[1::2]]
    w2b = jnp.concatenate(w2parts + [jnp.broadcast_to(b2, (_LANES,))])

    rpw = e_pad // (_IDS_PER_ROW * _NW)
    emb_rows = emb.shape[0]
    edge_fn = functools.partial(
        pl.kernel,
        out_type=jax.ShapeDtypeStruct((e_pad,), jnp.float32),
        mesh=plsc.VectorSubcoreMesh(
            core_axis_name="c", subcore_axis_name="s",
            num_cores=_N_CORES, num_subcores=_N_SUBCORES),
        scratch_types=[
            pltpu.VMEM((rpw, _IDS_PER_ROW), jnp.int32),
            pltpu.VMEM((rpw, _IDS_PER_ROW), jnp.int32),
            pltpu.VMEM((_NBUF, _CHUNK, h), jnp.bfloat16),
            pltpu.VMEM((_NBUF, _CHUNK, h), jnp.bfloat16),
            pltpu.VMEM((_NBUF, _CHUNK), jnp.float32),
            pltpu.VMEM((h + _LANES,), jnp.float32),
            pltpu.VMEM((_LANES * _LANES,), jnp.float32),
            pltpu.VMEM_SHARED((emb_rows, h), jnp.bfloat16),
            pltpu.VMEM_SHARED((emb_rows, h), jnp.bfloat16),
            pltpu.SemaphoreType.DMA((_NBUF,)),
            pltpu.SemaphoreType.DMA((_NBUF,)),
        ],
        compiler_params=pltpu.CompilerParams(
            needs_layout_passes=False, use_tc_tiling_on_sc=False),
    )(_edge_body)

    out = edge_fn(ps, pd, w2b, src2d, dst2d)
    return out[:e_total].reshape(e_total, 1)


# speedup vs baseline: 1.4441x; 1.0232x over previous
import functools

import jax
import jax.numpy as jnp
from jax import lax
from jax.experimental import pallas as pl
from jax.experimental.pallas import tpu as pltpu
from jax.experimental.pallas import tpu_sc as plsc

_N_CORES = 2
_N_SUBCORES = 16
_NW = _N_CORES * _N_SUBCORES
_LANES = 16

_G = 80
_NG = 5
_CHUNK = _G * _NG
_NBUF = 2


def _proj_body(emb_ref, w1a_ref, w1b_ref, b1_ref, ps_ref, pd_ref):
    e = emb_ref[...]
    ps_ref[...] = (
        jnp.dot(e, w1a_ref[...], preferred_element_type=jnp.float32) + b1_ref[...]
    ).astype(jnp.bfloat16)
    pd_ref[...] = jnp.dot(
        e, w1b_ref[...], preferred_element_type=jnp.float32).astype(jnp.bfloat16)


def _node_projections(emb, W1, b1):
    n, d = emb.shape
    h = W1.shape[1]
    return pl.pallas_call(
        _proj_body,
        out_shape=(
            jax.ShapeDtypeStruct((n, h), jnp.bfloat16),
            jax.ShapeDtypeStruct((n, h), jnp.bfloat16),
        ),
    )(emb, W1[:d], W1[d:], b1.reshape(1, h))


def _edge_body(ps_hbm, pd_hbm, w2b_hbm, src_hbm, dst_hbm, out_hbm,
               sidx_all, didx_all, srows, drows, outv, w2v, tbuf, ps_sh, pd_sh,
               gsems, osems):
    sid = lax.axis_index("s")
    wid = sid * _N_CORES + lax.axis_index("c")
    epw = src_hbm.shape[0] // _NW
    nchunks = epw // _CHUNK
    h = ps_hbm.shape[1]
    nk = h // (2 * _LANES)
    n_nodes = ps_hbm.shape[0]
    stripe = n_nodes // _N_SUBCORES
    pltpu.sync_copy(ps_hbm.at[pl.ds(sid * stripe, stripe)],
                    ps_sh.at[pl.ds(sid * stripe, stripe)])
    pltpu.sync_copy(pd_hbm.at[pl.ds(sid * stripe, stripe)],
                    pd_sh.at[pl.ds(sid * stripe, stripe)])
    pltpu.sync_copy(src_hbm.at[pl.ds(wid * epw, epw)], sidx_all)
    pltpu.sync_copy(dst_hbm.at[pl.ds(wid * epw, epw)], didx_all)
    pltpu.sync_copy(w2b_hbm, w2v)
    plsc.subcore_barrier()

    w2k = [w2v[pl.ds(_LANES * k, _LANES)] for k in range(h // _LANES)]
    w2pk = [plsc.pack(w2k[2 * k], w2k[2 * k + 1],
                      format=plsc.PackFormat.INTERLEAVED) for k in range(nk)]
    b2vec = w2v[pl.ds(h, _LANES)]
    col_idx = [lax.iota(jnp.int32, _LANES) * _LANES + l for l in range(_LANES)]

    def fetch(c, b):
        for j in range(_NG):
            idx = pl.ds(c * _CHUNK + j * _G, _G)
            dst_sl = pl.ds(j * _G, _G)
            pltpu.async_copy(ps_sh.at[sidx_all.at[idx]], srows.at[b, dst_sl],
                             gsems.at[b])
            pltpu.async_copy(pd_sh.at[didx_all.at[idx]], drows.at[b, dst_sl],
                             gsems.at[b])

    def wait_gathers(b):
        for _ in range(2 * _NG):
            pltpu.make_async_copy(
                ps_hbm.at[pl.ds(0, _G)],
                srows.at[b, pl.ds(0, _G)], gsems.at[b]).wait()

    def compute(c, b, out_pending):
        def group_body(g, gcarry):
            for q in range(_LANES // 4):
                accs = [None] * 4
                for k in range(nk):
                    for i in range(4):
                        e = g * _LANES + q * 4 + i
                        s = srows[b, e, pl.ds(2 * _LANES * k, 2 * _LANES)]
                        dd = drows[b, e, pl.ds(2 * _LANES * k, 2 * _LANES)]
                        t = jnp.maximum(s + dd, 0.0) * w2pk[k]
                        accs[i] = t if k == 0 else accs[i] + t
                for i in range(4):
                    pa, pb = plsc.unpack(
                        accs[i], format=plsc.PackFormat.INTERLEAVED,
                        preferred_element_type=jnp.float32)
                    tbuf[pl.ds((q * 4 + i) * _LANES, _LANES)] = pa + pb
            res = b2vec
            for l in range(_LANES):
                res = res + plsc.load_gather(tbuf, [col_idx[l]])
            outv[b, pl.ds(g * _LANES, _LANES)] = res
            return gcarry

        @pl.when(out_pending)
        def _():
            pltpu.make_async_copy(
                outv.at[b], out_hbm.at[pl.ds(0, _CHUNK)], osems.at[b]).wait()

        lax.fori_loop(0, _CHUNK // _LANES, group_body, 0)
        pltpu.async_copy(outv.at[b],
                         out_hbm.at[pl.ds(wid * epw + c * _CHUNK, _CHUNK)],
                         osems.at[b])

    cmax = nchunks - 1
    fetch(0, 0)

    def pair_body(p, carry):
        c0 = 2 * p
        fetch(jnp.minimum(c0 + 1, cmax), 1)
        wait_gathers(0)
        compute(c0, 0, p > 0)
        fetch(jnp.minimum(c0 + 2, cmax), 0)
        wait_gathers(1)
        compute(jnp.minimum(c0 + 1, cmax), 1, p > 0)
        return carry

    lax.fori_loop(0, (nchunks + 1) // 2, pair_body, 0)
    for b in range(_NBUF):
        pltpu.make_async_copy(
            outv.at[b], out_hbm.at[pl.ds(0, _CHUNK)], osems.at[b]).wait()


def kernel(emb, edges_src_ids, edges_dst_ids, W1, b1, W2, b2):
    e_total = edges_src_ids.shape[0]
    h = W1.shape[1]

    ps, pd = _node_projections(emb, W1, b1)

    src = edges_src_ids.astype(jnp.int32)
    dst = edges_dst_ids.astype(jnp.int32)

    w2col = W2[:, 0]
    w2parts = []
    for k in range(h // (2 * _LANES)):
        blk = w2col[2 * _LANES * k:2 * _LANES * (k + 1)]
        w2parts += [blk[0::2], blk[1::2]]
    w2b = jnp.concatenate(w2parts + [jnp.broadcast_to(b2, (_LANES,))])

    epw = e_total // _NW
    emb_rows = emb.shape[0]
    edge_fn = functools.partial(
        pl.kernel,
        out_type=jax.ShapeDtypeStruct((e_total,), jnp.float32),
        mesh=plsc.VectorSubcoreMesh(
            core_axis_name="c", subcore_axis_name="s",
            num_cores=_N_CORES, num_subcores=_N_SUBCORES),
        scratch_types=[
            pltpu.VMEM((epw,), jnp.int32),
            pltpu.VMEM((epw,), jnp.int32),
            pltpu.VMEM((_NBUF, _CHUNK, h), jnp.bfloat16),
            pltpu.VMEM((_NBUF, _CHUNK, h), jnp.bfloat16),
            pltpu.VMEM((_NBUF, _CHUNK), jnp.float32),
            pltpu.VMEM((h + _LANES,), jnp.float32),
            pltpu.VMEM((_LANES * _LANES,), jnp.float32),
            pltpu.VMEM_SHARED((emb_rows, h), jnp.bfloat16),
            pltpu.VMEM_SHARED((emb_rows, h), jnp.bfloat16),
            pltpu.SemaphoreType.DMA((_NBUF,)),
            pltpu.SemaphoreType.DMA((_NBUF,)),
        ],
        compiler_params=pltpu.CompilerParams(
            needs_layout_passes=False, use_tc_tiling_on_sc=False),
    )(_edge_body)

    out = edge_fn(ps, pd, w2b, src, dst)
    return out.reshape(e_total, 1)
